# baseline (speedup 1.0000x reference)
import functools
import jax
import jax.numpy as jnp
from jax import lax
from jax.experimental import pallas as pl
from jax.experimental.pallas import tpu as pltpu
from jax.experimental.pallas import tpu_sc as plsc

N = 10000
NP = 10240
E = 320000
C = 128
VOCAB = 100000
OUT = 16

NC = 2
NS = 16
NW = NC * NS
ECH = 128
EP = NW * 80 * ECH
EPT = EP // NW
NCH = EPT // ECH
RPT = NP // NS
DPT = NP // NS
XE = 80
XCH = 4
GCH = 20
NG = NCH // GCH

_sc_mesh = plsc.VectorSubcoreMesh(core_axis_name="c", subcore_axis_name="s")


@functools.partial(
    pl.kernel,
    out_type=(
        jax.ShapeDtypeStruct((NP, C), jnp.float32),
        jax.ShapeDtypeStruct((NC * NP,), jnp.float32),
        jax.ShapeDtypeStruct((NC * NP,), jnp.float32),
    ),
    mesh=_sc_mesh,
    scratch_types=[
        pltpu.VMEM((XCH, XE), jnp.int32),
        pltpu.VMEM((NG, GCH, ECH), jnp.int32),
        pltpu.VMEM((NG, GCH, ECH), jnp.int32),
        pltpu.VMEM((XE, C), jnp.float32),
        pltpu.VMEM((ECH,), jnp.float32),
        pltpu.VMEM_SHARED((NP,), jnp.float32),
        pltpu.VMEM_SHARED((NP,), jnp.float32),
        pltpu.SemaphoreType.DMA,
        pltpu.SemaphoreType.DMA,
    ],
)
def _sc_embed_deg(xp_hbm, src_hbm, dst_hbm, table_hbm, zeros1_hbm, ones_hbm,
                  h_out, degp_out, degtp_out,
                  xidx, esrc, edst, rows, ones, acc_d, acc_t, sem, semh):
    c = lax.axis_index("c")
    s = lax.axis_index("s")
    wid = c * NS + s

    pltpu.sync_copy(src_hbm.at[wid], esrc)
    pltpu.sync_copy(dst_hbm.at[wid], edst)
    pltpu.sync_copy(xp_hbm.at[wid], xidx)
    pltpu.sync_copy(ones_hbm, ones)
    pltpu.sync_copy(zeros1_hbm.at[pl.ds(s * DPT, DPT)], acc_d.at[pl.ds(s * DPT, DPT)])
    pltpu.sync_copy(zeros1_hbm.at[pl.ds(s * DPT, DPT)], acc_t.at[pl.ds(s * DPT, DPT)])
    plsc.subcore_barrier()

    def _hist_g(g, carry):
        def _hist(j, carry2):
            pltpu.async_copy(ones, acc_d.at[edst.at[g, j]], semh, add=True)
            pltpu.async_copy(ones, acc_t.at[esrc.at[g, j]], semh, add=True)
            return carry2

        lax.fori_loop(0, GCH, _hist, 0)
        return carry

    lax.fori_loop(0, NG, _hist_g, 0)

    def _emb(j, carry):
        pltpu.async_copy(table_hbm.at[xidx.at[j]], rows, sem).wait()
        base = wid * (XCH * XE) + j * XE
        pltpu.sync_copy(rows, h_out.at[pl.ds(base, XE)])
        return carry

    lax.fori_loop(0, XCH, _emb, 0)

    def _drain(j, carry):
        pltpu.make_async_copy(ones, acc_d.at[edst.at[0, 0]], semh).wait()
        pltpu.make_async_copy(ones, acc_t.at[esrc.at[0, 0]], semh).wait()
        return carry

    lax.fori_loop(0, NG * GCH, _drain, 0)

    plsc.subcore_barrier()
    base = c * NP + s * DPT
    pltpu.sync_copy(acc_d.at[pl.ds(s * DPT, DPT)], degp_out.at[pl.ds(base, DPT)])
    pltpu.sync_copy(acc_t.at[pl.ds(s * DPT, DPT)], degtp_out.at[pl.ds(base, DPT)])


@functools.partial(
    pl.kernel,
    out_type=(jax.ShapeDtypeStruct((NP, C), jnp.float32),
              jax.ShapeDtypeStruct((NP, C), jnp.float32)),
    mesh=_sc_mesh,
    scratch_types=[
        pltpu.VMEM((GCH, ECH), jnp.int32),
        pltpu.VMEM((GCH, ECH), jnp.int32),
        pltpu.VMEM((ECH, C), jnp.float32),
        pltpu.VMEM((ECH, C), jnp.float32),
        pltpu.VMEM_SHARED((NP, C), jnp.float32),
        pltpu.SemaphoreType.DMA,
        pltpu.SemaphoreType.DMA,
        pltpu.SemaphoreType.DMA,
        pltpu.SemaphoreType.DMA,
    ],
)
def _sc_agg2(gin_hbm, gout_hbm, src0_hbm, dst0_hbm, srcN_hbm, dstN_hbm,
             zeros_hbm, aggin_out, aggout_out,
             esrc, edst, rows0, rows1, acc, sem0, sem1, sems0, sems1):
    c = lax.axis_index("c")
    s = lax.axis_index("s")

    pltpu.sync_copy(zeros_hbm.at[pl.ds(s * RPT, RPT)], acc.at[pl.ds(s * RPT, RPT)])
    plsc.subcore_barrier()

    def _run(g_hbm, gat_hbm, sca_hbm):
        def _gstart(j, rows, sem):
            pltpu.async_copy(g_hbm.at[esrc.at[j]], rows, sem)

        def _gwait(rows, sem):
            pltpu.make_async_copy(g_hbm.at[esrc.at[0]], rows, sem).wait()

        def _sstart(j, rows, sem):
            pltpu.async_copy(rows, acc.at[edst.at[j]], sem, add=True)

        def _swait(rows, sem):
            pltpu.make_async_copy(rows, acc.at[edst.at[0]], sem).wait()

        def _slice(t, carry0):
            wid = t * NS + s

            def _group(g, carry):
                pltpu.sync_copy(gat_hbm.at[wid, g], esrc)
                pltpu.sync_copy(sca_hbm.at[wid, g], edst)
                _gstart(0, rows0, sem0)
                _gstart(1, rows1, sem1)

                def _body(i, carry2):
                    j0 = 2 * i
                    _gwait(rows0, sem0)
                    _sstart(j0, rows0, sems0)
                    _gwait(rows1, sem1)
                    _sstart(j0 + 1, rows1, sems1)

                    @pl.when(j0 + 2 < GCH)
                    def _():
                        _swait(rows0, sems0)
                        _gstart(j0 + 2, rows0, sem0)

                    @pl.when(j0 + 3 < GCH)
                    def _():
                        _swait(rows1, sems1)
                        _gstart(j0 + 3, rows1, sem1)

                    return carry2

                lax.fori_loop(0, GCH // 2, _body, 0)
                _swait(rows0, sems0)
                _swait(rows1, sems1)
                return carry

            lax.fori_loop(0, NG, _group, 0)
            return carry0

        lax.fori_loop(0, NC, _slice, 0)

    @pl.when(c == 0)
    def _():
        _run(gin_hbm, src0_hbm, dstN_hbm)

    @pl.when(c == 1)
    def _():
        _run(gout_hbm, dst0_hbm, srcN_hbm)

    plsc.subcore_barrier()

    @pl.when(c == 0)
    def _():
        pltpu.sync_copy(acc.at[pl.ds(s * RPT, RPT)],
                        aggin_out.at[pl.ds(s * RPT, RPT)])

    @pl.when(c == 1)
    def _():
        pltpu.sync_copy(acc.at[pl.ds(s * RPT, RPT)],
                        aggout_out.at[pl.ds(s * RPT, RPT)])


BN = 1000
GRID = N // BN


def _mm(a, b):
    return jnp.dot(a, b, preferred_element_type=jnp.float32)


def _branch(cat, wl_ref, bl_ref, wc_ref, dis):
    acc = _mm(cat[0], wl_ref[0:C, :])
    for k in range(1, len(cat)):
        acc += _mm(cat[k], wl_ref[k * C:(k + 1) * C, :])
    p = jnp.maximum(acc + bl_ref[...], 0.0)
    return _mm(p, wc_ref[...]) * dis


def _fused0_body(h_ref, wli_ref, bli_ref, wci_ref, dis_ref,
                 wlo_ref, blo_ref, wco_ref, dist_ref, gi_ref, go_ref):
    cat = (h_ref[...],)
    gi_ref[...] = _branch(cat, wli_ref, bli_ref, wci_ref, dis_ref[...])
    go_ref[...] = _branch(cat, wlo_ref, blo_ref, wco_ref, dist_ref[...])


_wspec = [
    pl.BlockSpec((C, C), lambda i: (0, 0)),
    pl.BlockSpec((1, C), lambda i: (0, 0)),
    pl.BlockSpec((C, C), lambda i: (0, 0)),
    pl.BlockSpec((BN, 1), lambda i: (i, 0)),
]
_nspec = pl.BlockSpec((BN, C), lambda i: (i, 0))

_fused0 = pl.pallas_call(
    _fused0_body,
    grid=(GRID,),
    in_specs=[_nspec] + _wspec + _wspec,
    out_specs=(_nspec, _nspec),
    out_shape=(jax.ShapeDtypeStruct((N, C), jnp.float32),
               jax.ShapeDtypeStruct((N, C), jnp.float32)),
)


def _fused1_body(h_ref, ai_ref, gi0_ref, bci_ref, dis_ref,
                 ao_ref, go0_ref, bco_ref, dist_ref,
                 wli_ref, bli_ref, wci_ref, wlo_ref, blo_ref, wco_ref,
                 gi_ref, go_ref, xi_ref, xo_ref):
    dis = dis_ref[...]
    dist = dist_ref[...]
    x_in0 = jnp.maximum(
        (ai_ref[...] + gi0_ref[...]) * dis + bci_ref[...], 0.0)
    x_out0 = jnp.maximum(
        (ao_ref[...] + go0_ref[...]) * dist + bco_ref[...], 0.0)
    xi_ref[...] = x_in0
    xo_ref[...] = x_out0
    cat = (h_ref[...], x_in0, x_out0)
    gi_ref[...] = _branch(cat, wli_ref, bli_ref, wci_ref, dis)
    go_ref[...] = _branch(cat, wlo_ref, blo_ref, wco_ref, dist)


_w3spec = [
    pl.BlockSpec((3 * C, C), lambda i: (0, 0)),
    pl.BlockSpec((1, C), lambda i: (0, 0)),
    pl.BlockSpec((C, C), lambda i: (0, 0)),
]

_fused1 = pl.pallas_call(
    _fused1_body,
    grid=(GRID,),
    in_specs=([_nspec,
               _nspec, _nspec, pl.BlockSpec((1, C), lambda i: (0, 0)),
               pl.BlockSpec((BN, 1), lambda i: (i, 0)),
               _nspec, _nspec, pl.BlockSpec((1, C), lambda i: (0, 0)),
               pl.BlockSpec((BN, 1), lambda i: (i, 0))]
              + _w3spec + _w3spec),
    out_specs=(_nspec, _nspec, _nspec, _nspec),
    out_shape=tuple(jax.ShapeDtypeStruct((N, C), jnp.float32)
                    for _ in range(4)),
)


def _dis_body(degp_ref, degtp_ref, dis_ref, dist_ref):
    d = degp_ref[0:1, :] + degp_ref[1:2, :] + 1.0
    dis_ref[...] = lax.rsqrt(d)
    dt = degtp_ref[0:1, :] + degtp_ref[1:2, :] + 1.0
    dist_ref[...] = lax.rsqrt(dt)


_dis_call = pl.pallas_call(
    _dis_body,
    out_shape=(jax.ShapeDtypeStruct((1, NP), jnp.float32),
               jax.ShapeDtypeStruct((1, NP), jnp.float32)),
)


def _colsum_body(x0, x1, x2, ai_ref, gi_ref, bci_ref, dis_ref,
                 ao_ref, go_ref, bco_ref, dist_ref, out_ref):
    i = pl.program_id(0)

    @pl.when(i == 0)
    def _():
        out_ref[...] = jnp.zeros_like(out_ref)

    x_in1 = jnp.maximum(
        (ai_ref[...] + gi_ref[...]) * dis_ref[...] + bci_ref[...], 0.0)
    x_out1 = jnp.maximum(
        (ao_ref[...] + go_ref[...]) * dist_ref[...] + bco_ref[...], 0.0)
    for k, x in enumerate((x0[...], x1[...], x2[...], x_in1, x_out1)):
        out_ref[:, k * C:(k + 1) * C] += jnp.sum(x, axis=0, keepdims=True)


_colsum = pl.pallas_call(
    _colsum_body,
    grid=(GRID,),
    in_specs=[_nspec, _nspec, _nspec,
              _nspec, _nspec, pl.BlockSpec((1, C), lambda i: (0, 0)),
              pl.BlockSpec((BN, 1), lambda i: (i, 0)),
              _nspec, _nspec, pl.BlockSpec((1, C), lambda i: (0, 0)),
              pl.BlockSpec((BN, 1), lambda i: (i, 0))],
    out_specs=pl.BlockSpec((1, 5 * C), lambda i: (0, 0)),
    out_shape=jax.ShapeDtypeStruct((1, 5 * C), jnp.float32),
)


def _mlp_body(feat_ref, hw_ref, hb_ref, mw_ref, mb_ref, vw_ref, vb_ref,
              mean_ref, var_ref):
    f = feat_ref[...] * (1.0 / N)
    hdn = jnp.maximum(
        jnp.dot(f, hw_ref[...], preferred_element_type=jnp.float32)
        + hb_ref[...], 0.0)
    m = jnp.dot(hdn, mw_ref[...], preferred_element_type=jnp.float32) + mb_ref[...]
    mean_ref[...] = 2.0 * jnp.tanh(m)
    v = jnp.dot(hdn, vw_ref[...], preferred_element_type=jnp.float32) + vb_ref[...]
    var_ref[...] = 2.0 * jax.nn.sigmoid(v)


_mlp = pl.pallas_call(
    _mlp_body,
    out_shape=(jax.ShapeDtypeStruct((1, OUT), jnp.float32),
               jax.ShapeDtypeStruct((1, OUT), jnp.float32)),
)


def kernel(x, edge_index, edge_index_t, embed_W,
           li_W0, li_b0, lo_W0, lo_b0, ci_W0, ci_b0, co_W0, co_b0,
           li_W1, li_b1, lo_W1, lo_b1, ci_W1, ci_b1, co_W1, co_b1,
           hidden_W, hidden_b, mean_W, mean_b, var_W, var_b):
    npad = EP - E

    pad_scatter = N + jnp.arange(npad, dtype=jnp.int32) % (NP - N)
    pad_gather = jnp.arange(npad, dtype=jnp.int32) % N

    def pad_edges(ids, fill):
        return jnp.concatenate([ids, fill]).reshape(NW, NG, GCH, ECH)

    src0 = pad_edges(edge_index[0], pad_gather)
    dst0 = pad_edges(edge_index[1], pad_gather)
    srcN = pad_edges(edge_index[0], pad_scatter)
    dstN = pad_edges(edge_index[1], pad_scatter)
    xp = jnp.concatenate(
        [x, jnp.zeros((NP - N,), jnp.int32)]).reshape(NW, XCH, XE)
    zeros1 = jnp.zeros((NP,), jnp.float32)
    ones1 = jnp.ones((ECH,), jnp.float32)
    zeros2 = jnp.zeros((NP, C), jnp.float32)

    hp, degp, degtp = _sc_embed_deg(xp, srcN, dstN, embed_W, zeros1, ones1)
    h = hp[:N]

    dis2, dist2 = _dis_call(degp.reshape(NC, NP), degtp.reshape(NC, NP))
    dis = dis2.reshape(NP, 1)[:N]
    dist = dist2.reshape(NP, 1)[:N]

    def b2(b):
        return b.reshape(1, -1)

    g_in0, g_out0 = _fused0(h, li_W0, b2(li_b0), ci_W0, dis,
                            lo_W0, b2(lo_b0), co_W0, dist)
    agg_in0, agg_out0 = _sc_agg2(g_in0, g_out0, src0, dst0, srcN, dstN, zeros2)

    g_in1, g_out1, x_in0, x_out0 = _fused1(
        h, agg_in0, g_in0, b2(ci_b0), dis,
        agg_out0, g_out0, b2(co_b0), dist,
        li_W1, b2(li_b1), ci_W1, lo_W1, b2(lo_b1), co_W1)
    agg_in1, agg_out1 = _sc_agg2(g_in1, g_out1, src0, dst0, srcN, dstN, zeros2)

    feat = _colsum(h, x_in0, x_out0,
                   agg_in1, g_in1, b2(ci_b1), dis,
                   agg_out1, g_out1, b2(co_b1), dist)
    mean, var = _mlp(feat, hidden_W, b2(hidden_b), mean_W, b2(mean_b),
                     var_W, b2(var_b))
    return (mean.reshape(OUT), var.reshape(OUT))


# speedup vs baseline: 1.2135x; 1.2135x over previous
import functools
import jax
import jax.numpy as jnp
from jax import lax
from jax.experimental import pallas as pl
from jax.experimental.pallas import tpu as pltpu
from jax.experimental.pallas import tpu_sc as plsc

N = 10000
NP = 10240
E = 320000
C = 128
VOCAB = 100000
OUT = 16

NC = 2
NS = 16
NW = NC * NS
ECH = 128
EP = NW * 80 * ECH
EPT = EP // NW
NCH = EPT // ECH
RPT = NP // NS
DPT = NP // NS
XE = 80
XCH = 4
GCH = 20
NG = NCH // GCH

_sc_mesh = plsc.VectorSubcoreMesh(core_axis_name="c", subcore_axis_name="s")


@functools.partial(
    pl.kernel,
    out_type=(
        jax.ShapeDtypeStruct((NP, C), jnp.float32),
        jax.ShapeDtypeStruct((NC * NP,), jnp.float32),
        jax.ShapeDtypeStruct((NC * NP,), jnp.float32),
    ),
    mesh=_sc_mesh,
    scratch_types=[
        pltpu.VMEM((XCH, XE), jnp.int32),
        pltpu.VMEM((NG, GCH, ECH), jnp.int32),
        pltpu.VMEM((NG, GCH, ECH), jnp.int32),
        pltpu.VMEM((XE, C), jnp.float32),
        pltpu.VMEM((ECH,), jnp.float32),
        pltpu.VMEM_SHARED((NP,), jnp.float32),
        pltpu.VMEM_SHARED((NP,), jnp.float32),
        pltpu.SemaphoreType.DMA,
    ],
)
def _sc_embed_deg(xp_hbm, src_hbm, dst_hbm, table_hbm, zeros1_hbm, ones_hbm,
                  h_out, degp_out, degtp_out,
                  xidx, esrc, edst, rows, ones, acc_d, acc_t, sem):
    c = lax.axis_index("c")
    s = lax.axis_index("s")
    wid = c * NS + s

    pltpu.sync_copy(src_hbm.at[wid], esrc)
    pltpu.sync_copy(dst_hbm.at[wid], edst)
    pltpu.sync_copy(xp_hbm.at[wid], xidx)
    pltpu.sync_copy(ones_hbm, ones)
    pltpu.sync_copy(zeros1_hbm.at[pl.ds(s * DPT, DPT)], acc_d.at[pl.ds(s * DPT, DPT)])
    pltpu.sync_copy(zeros1_hbm.at[pl.ds(s * DPT, DPT)], acc_t.at[pl.ds(s * DPT, DPT)])
    plsc.subcore_barrier()

    def _hist_g(g, carry):
        def _hist(j, carry2):
            pltpu.sync_copy(ones, acc_d.at[edst.at[g, j]], add=True)
            pltpu.sync_copy(ones, acc_t.at[esrc.at[g, j]], add=True)
            return carry2

        lax.fori_loop(0, GCH, _hist, 0)
        return carry

    lax.fori_loop(0, NG, _hist_g, 0)

    def _emb(j, carry):
        pltpu.async_copy(table_hbm.at[xidx.at[j]], rows, sem).wait()
        base = wid * (XCH * XE) + j * XE
        pltpu.sync_copy(rows, h_out.at[pl.ds(base, XE)])
        return carry

    lax.fori_loop(0, XCH, _emb, 0)

    plsc.subcore_barrier()
    base = c * NP + s * DPT
    pltpu.sync_copy(acc_d.at[pl.ds(s * DPT, DPT)], degp_out.at[pl.ds(base, DPT)])
    pltpu.sync_copy(acc_t.at[pl.ds(s * DPT, DPT)], degtp_out.at[pl.ds(base, DPT)])


@functools.partial(
    pl.kernel,
    out_type=(jax.ShapeDtypeStruct((NP, C), jnp.float32),
              jax.ShapeDtypeStruct((NP, C), jnp.float32)),
    mesh=_sc_mesh,
    scratch_types=[
        pltpu.VMEM((GCH, ECH), jnp.int32),
        pltpu.VMEM((GCH, ECH), jnp.int32),
        pltpu.VMEM((ECH, C), jnp.float32),
        pltpu.VMEM((ECH, C), jnp.float32),
        pltpu.VMEM_SHARED((NP, C), jnp.float32),
        pltpu.SemaphoreType.DMA,
        pltpu.SemaphoreType.DMA,
    ],
)
def _sc_agg2(gin_hbm, gout_hbm, src0_hbm, dst0_hbm, srcN_hbm, dstN_hbm,
             zeros_hbm, aggin_out, aggout_out,
             esrc, edst, rows0, rows1, acc, sem0, sem1):
    c = lax.axis_index("c")
    s = lax.axis_index("s")

    pltpu.sync_copy(zeros_hbm.at[pl.ds(s * RPT, RPT)], acc.at[pl.ds(s * RPT, RPT)])
    plsc.subcore_barrier()

    def _run(g_hbm, gat_hbm, sca_hbm):
        def _start(j, rows, sem):
            pltpu.async_copy(g_hbm.at[esrc.at[j]], rows, sem)

        def _wait(rows, sem):
            pltpu.make_async_copy(g_hbm.at[esrc.at[0]], rows, sem).wait()

        def _scat(j, rows):
            pltpu.sync_copy(rows, acc.at[edst.at[j]], add=True)

        def _slice(t, carry0):
            wid = t * NS + s

            def _group(g, carry):
                pltpu.sync_copy(gat_hbm.at[wid, g], esrc)
                pltpu.sync_copy(sca_hbm.at[wid, g], edst)
                _start(0, rows0, sem0)

                def _body(i, carry2):
                    j0 = 2 * i
                    _start(j0 + 1, rows1, sem1)
                    _wait(rows0, sem0)
                    _scat(j0, rows0)

                    @pl.when(j0 + 2 < GCH)
                    def _():
                        _start(j0 + 2, rows0, sem0)

                    _wait(rows1, sem1)
                    _scat(j0 + 1, rows1)
                    return carry2

                lax.fori_loop(0, GCH // 2, _body, 0)
                return carry

            lax.fori_loop(0, NG, _group, 0)
            return carry0

        lax.fori_loop(0, NC, _slice, 0)

    @pl.when(c == 0)
    def _():
        _run(gin_hbm, src0_hbm, dstN_hbm)

    @pl.when(c == 1)
    def _():
        _run(gout_hbm, dst0_hbm, srcN_hbm)

    plsc.subcore_barrier()

    @pl.when(c == 0)
    def _():
        pltpu.sync_copy(acc.at[pl.ds(s * RPT, RPT)],
                        aggin_out.at[pl.ds(s * RPT, RPT)])

    @pl.when(c == 1)
    def _():
        pltpu.sync_copy(acc.at[pl.ds(s * RPT, RPT)],
                        aggout_out.at[pl.ds(s * RPT, RPT)])


BN = 1000
GRID = N // BN


def _mm(a, b):
    return jnp.dot(a, b, preferred_element_type=jnp.float32)


def _branch(cat, wl_ref, bl_ref, wc_ref, dis):
    acc = _mm(cat[0], wl_ref[0:C, :])
    for k in range(1, len(cat)):
        acc += _mm(cat[k], wl_ref[k * C:(k + 1) * C, :])
    p = jnp.maximum(acc + bl_ref[...], 0.0)
    return _mm(p, wc_ref[...]) * dis


def _fused0_body(h_ref, wli_ref, bli_ref, wci_ref, dis_ref,
                 wlo_ref, blo_ref, wco_ref, dist_ref, gi_ref, go_ref):
    cat = (h_ref[...],)
    gi_ref[...] = _branch(cat, wli_ref, bli_ref, wci_ref, dis_ref[...])
    go_ref[...] = _branch(cat, wlo_ref, blo_ref, wco_ref, dist_ref[...])


_wspec = [
    pl.BlockSpec((C, C), lambda i: (0, 0)),
    pl.BlockSpec((1, C), lambda i: (0, 0)),
    pl.BlockSpec((C, C), lambda i: (0, 0)),
    pl.BlockSpec((BN, 1), lambda i: (i, 0)),
]
_nspec = pl.BlockSpec((BN, C), lambda i: (i, 0))

_fused0 = pl.pallas_call(
    _fused0_body,
    grid=(GRID,),
    in_specs=[_nspec] + _wspec + _wspec,
    out_specs=(_nspec, _nspec),
    out_shape=(jax.ShapeDtypeStruct((N, C), jnp.float32),
               jax.ShapeDtypeStruct((N, C), jnp.float32)),
)


def _fused1_body(h_ref, ai_ref, gi0_ref, bci_ref, dis_ref,
                 ao_ref, go0_ref, bco_ref, dist_ref,
                 wli_ref, bli_ref, wci_ref, wlo_ref, blo_ref, wco_ref,
                 gi_ref, go_ref, xi_ref, xo_ref):
    dis = dis_ref[...]
    dist = dist_ref[...]
    x_in0 = jnp.maximum(
        (ai_ref[...] + gi0_ref[...]) * dis + bci_ref[...], 0.0)
    x_out0 = jnp.maximum(
        (ao_ref[...] + go0_ref[...]) * dist + bco_ref[...], 0.0)
    xi_ref[...] = x_in0
    xo_ref[...] = x_out0
    cat = (h_ref[...], x_in0, x_out0)
    gi_ref[...] = _branch(cat, wli_ref, bli_ref, wci_ref, dis)
    go_ref[...] = _branch(cat, wlo_ref, blo_ref, wco_ref, dist)


_w3spec = [
    pl.BlockSpec((3 * C, C), lambda i: (0, 0)),
    pl.BlockSpec((1, C), lambda i: (0, 0)),
    pl.BlockSpec((C, C), lambda i: (0, 0)),
]

_fused1 = pl.pallas_call(
    _fused1_body,
    grid=(GRID,),
    in_specs=([_nspec,
               _nspec, _nspec, pl.BlockSpec((1, C), lambda i: (0, 0)),
               pl.BlockSpec((BN, 1), lambda i: (i, 0)),
               _nspec, _nspec, pl.BlockSpec((1, C), lambda i: (0, 0)),
               pl.BlockSpec((BN, 1), lambda i: (i, 0))]
              + _w3spec + _w3spec),
    out_specs=(_nspec, _nspec, _nspec, _nspec),
    out_shape=tuple(jax.ShapeDtypeStruct((N, C), jnp.float32)
                    for _ in range(4)),
)


def _dis_body(degp_ref, degtp_ref, dis_ref, dist_ref):
    d = degp_ref[0:1, :] + degp_ref[1:2, :] + 1.0
    dis_ref[...] = lax.rsqrt(d)
    dt = degtp_ref[0:1, :] + degtp_ref[1:2, :] + 1.0
    dist_ref[...] = lax.rsqrt(dt)


_dis_call = pl.pallas_call(
    _dis_body,
    out_shape=(jax.ShapeDtypeStruct((1, NP), jnp.float32),
               jax.ShapeDtypeStruct((1, NP), jnp.float32)),
)


def _colsum_body(x0, x1, x2, ai_ref, gi_ref, bci_ref, dis_ref,
                 ao_ref, go_ref, bco_ref, dist_ref, out_ref):
    i = pl.program_id(0)

    @pl.when(i == 0)
    def _():
        out_ref[...] = jnp.zeros_like(out_ref)

    x_in1 = jnp.maximum(
        (ai_ref[...] + gi_ref[...]) * dis_ref[...] + bci_ref[...], 0.0)
    x_out1 = jnp.maximum(
        (ao_ref[...] + go_ref[...]) * dist_ref[...] + bco_ref[...], 0.0)
    for k, x in enumerate((x0[...], x1[...], x2[...], x_in1, x_out1)):
        out_ref[:, k * C:(k + 1) * C] += jnp.sum(x, axis=0, keepdims=True)


_colsum = pl.pallas_call(
    _colsum_body,
    grid=(GRID,),
    in_specs=[_nspec, _nspec, _nspec,
              _nspec, _nspec, pl.BlockSpec((1, C), lambda i: (0, 0)),
              pl.BlockSpec((BN, 1), lambda i: (i, 0)),
              _nspec, _nspec, pl.BlockSpec((1, C), lambda i: (0, 0)),
              pl.BlockSpec((BN, 1), lambda i: (i, 0))],
    out_specs=pl.BlockSpec((1, 5 * C), lambda i: (0, 0)),
    out_shape=jax.ShapeDtypeStruct((1, 5 * C), jnp.float32),
)


def _mlp_body(feat_ref, hw_ref, hb_ref, mw_ref, mb_ref, vw_ref, vb_ref,
              mean_ref, var_ref):
    f = feat_ref[...] * (1.0 / N)
    hdn = jnp.maximum(
        jnp.dot(f, hw_ref[...], preferred_element_type=jnp.float32)
        + hb_ref[...], 0.0)
    m = jnp.dot(hdn, mw_ref[...], preferred_element_type=jnp.float32) + mb_ref[...]
    mean_ref[...] = 2.0 * jnp.tanh(m)
    v = jnp.dot(hdn, vw_ref[...], preferred_element_type=jnp.float32) + vb_ref[...]
    var_ref[...] = 2.0 * jax.nn.sigmoid(v)


_mlp = pl.pallas_call(
    _mlp_body,
    out_shape=(jax.ShapeDtypeStruct((1, OUT), jnp.float32),
               jax.ShapeDtypeStruct((1, OUT), jnp.float32)),
)


def kernel(x, edge_index, edge_index_t, embed_W,
           li_W0, li_b0, lo_W0, lo_b0, ci_W0, ci_b0, co_W0, co_b0,
           li_W1, li_b1, lo_W1, lo_b1, ci_W1, ci_b1, co_W1, co_b1,
           hidden_W, hidden_b, mean_W, mean_b, var_W, var_b):
    npad = EP - E

    pad_scatter = N + jnp.arange(npad, dtype=jnp.int32) % (NP - N)
    pad_gather = jnp.arange(npad, dtype=jnp.int32) % N

    def pad_edges(ids, fill):
        return jnp.concatenate([ids, fill]).reshape(NW, NG, GCH, ECH)

    src0 = pad_edges(edge_index[0], pad_gather)
    dst0 = pad_edges(edge_index[1], pad_gather)
    srcN = pad_edges(edge_index[0], pad_scatter)
    dstN = pad_edges(edge_index[1], pad_scatter)
    xp = jnp.concatenate(
        [x, jnp.zeros((NP - N,), jnp.int32)]).reshape(NW, XCH, XE)
    zeros1 = jnp.zeros((NP,), jnp.float32)
    ones1 = jnp.ones((ECH,), jnp.float32)
    zeros2 = jnp.zeros((NP, C), jnp.float32)

    hp, degp, degtp = _sc_embed_deg(xp, srcN, dstN, embed_W, zeros1, ones1)
    h = hp[:N]

    dis2, dist2 = _dis_call(degp.reshape(NC, NP), degtp.reshape(NC, NP))
    dis = dis2.reshape(NP, 1)[:N]
    dist = dist2.reshape(NP, 1)[:N]

    def b2(b):
        return b.reshape(1, -1)

    g_in0, g_out0 = _fused0(h, li_W0, b2(li_b0), ci_W0, dis,
                            lo_W0, b2(lo_b0), co_W0, dist)
    agg_in0, agg_out0 = _sc_agg2(g_in0, g_out0, src0, dst0, srcN, dstN, zeros2)

    g_in1, g_out1, x_in0, x_out0 = _fused1(
        h, agg_in0, g_in0, b2(ci_b0), dis,
        agg_out0, g_out0, b2(co_b0), dist,
        li_W1, b2(li_b1), ci_W1, lo_W1, b2(lo_b1), co_W1)
    agg_in1, agg_out1 = _sc_agg2(g_in1, g_out1, src0, dst0, srcN, dstN, zeros2)

    feat = _colsum(h, x_in0, x_out0,
                   agg_in1, g_in1, b2(ci_b1), dis,
                   agg_out1, g_out1, b2(co_b1), dist)
    mean, var = _mlp(feat, hidden_W, b2(hidden_b), mean_W, b2(mean_b),
                     var_W, b2(var_b))
    return (mean.reshape(OUT), var.reshape(OUT))


# speedup vs baseline: 1.2841x; 1.0582x over previous
import functools
import jax
import jax.numpy as jnp
from jax import lax
from jax.experimental import pallas as pl
from jax.experimental.pallas import tpu as pltpu
from jax.experimental.pallas import tpu_sc as plsc

N = 10000
NP = 10240
E = 320000
C = 128
VOCAB = 100000
OUT = 16

NC = 2
NS = 16
NW = NC * NS
ECH = 128
EP = NW * 80 * ECH
EPT = EP // NW
NCH = EPT // ECH
RPT = NP // NS
DPT = NP // NS
XE = 80
XCH = 4
GCH = 40
NG = NCH // GCH

_sc_mesh = plsc.VectorSubcoreMesh(core_axis_name="c", subcore_axis_name="s")


@functools.partial(
    pl.kernel,
    out_type=(
        jax.ShapeDtypeStruct((NP, C), jnp.float32),
        jax.ShapeDtypeStruct((NC * NP,), jnp.float32),
        jax.ShapeDtypeStruct((NC * NP,), jnp.float32),
    ),
    mesh=_sc_mesh,
    scratch_types=[
        pltpu.VMEM((XCH, XE), jnp.int32),
        pltpu.VMEM((NG, GCH, ECH), jnp.int32),
        pltpu.VMEM((NG, GCH, ECH), jnp.int32),
        pltpu.VMEM((XE, C), jnp.float32),
        pltpu.VMEM((ECH,), jnp.float32),
        pltpu.VMEM_SHARED((NP,), jnp.float32),
        pltpu.VMEM_SHARED((NP,), jnp.float32),
        pltpu.SemaphoreType.DMA,
    ],
)
def _sc_embed_deg(xp_hbm, src_hbm, dst_hbm, table_hbm, zeros1_hbm, ones_hbm,
                  h_out, degp_out, degtp_out,
                  xidx, esrc, edst, rows, ones, acc_d, acc_t, sem):
    c = lax.axis_index("c")
    s = lax.axis_index("s")
    wid = c * NS + s

    pltpu.sync_copy(src_hbm.at[wid], esrc)
    pltpu.sync_copy(dst_hbm.at[wid], edst)
    pltpu.sync_copy(xp_hbm.at[wid], xidx)
    pltpu.sync_copy(ones_hbm, ones)
    pltpu.sync_copy(zeros1_hbm.at[pl.ds(s * DPT, DPT)], acc_d.at[pl.ds(s * DPT, DPT)])
    pltpu.sync_copy(zeros1_hbm.at[pl.ds(s * DPT, DPT)], acc_t.at[pl.ds(s * DPT, DPT)])
    plsc.subcore_barrier()

    def _hist_g(g, carry):
        def _hist(j, carry2):
            pltpu.sync_copy(ones, acc_d.at[edst.at[g, j]], add=True)
            pltpu.sync_copy(ones, acc_t.at[esrc.at[g, j]], add=True)
            return carry2

        lax.fori_loop(0, GCH, _hist, 0)
        return carry

    lax.fori_loop(0, NG, _hist_g, 0)

    def _emb(j, carry):
        pltpu.async_copy(table_hbm.at[xidx.at[j]], rows, sem).wait()
        base = wid * (XCH * XE) + j * XE
        pltpu.sync_copy(rows, h_out.at[pl.ds(base, XE)])
        return carry

    lax.fori_loop(0, XCH, _emb, 0)

    plsc.subcore_barrier()
    base = c * NP + s * DPT
    pltpu.sync_copy(acc_d.at[pl.ds(s * DPT, DPT)], degp_out.at[pl.ds(base, DPT)])
    pltpu.sync_copy(acc_t.at[pl.ds(s * DPT, DPT)], degtp_out.at[pl.ds(base, DPT)])


@functools.partial(
    pl.kernel,
    out_type=(jax.ShapeDtypeStruct((NP, C), jnp.float32),
              jax.ShapeDtypeStruct((NP, C), jnp.float32)),
    mesh=_sc_mesh,
    scratch_types=[
        pltpu.VMEM((GCH, ECH), jnp.int32),
        pltpu.VMEM((GCH, ECH), jnp.int32),
        pltpu.VMEM((ECH, C), jnp.float32),
        pltpu.VMEM((ECH, C), jnp.float32),
        pltpu.VMEM_SHARED((NP, C), jnp.float32),
        pltpu.SemaphoreType.DMA,
        pltpu.SemaphoreType.DMA,
    ],
)
def _sc_agg2(gin_hbm, gout_hbm, src0_hbm, dst0_hbm, srcN_hbm, dstN_hbm,
             aggin_out, aggout_out,
             esrc, edst, rows0, rows1, acc, sem0, sem1):
    c = lax.axis_index("c")
    s = lax.axis_index("s")

    def _z(i, carry):
        def _zk(k, carry2):
            rows0[i, pl.ds(k * 16, 16)] = jnp.zeros((16,), jnp.float32)
            return carry2

        lax.fori_loop(0, C // 16, _zk, 0)
        return carry

    lax.fori_loop(0, ECH, _z, 0)

    def _zs(i, carry):
        pltpu.sync_copy(rows0, acc.at[pl.ds(s * RPT + i * ECH, ECH)])
        return carry

    lax.fori_loop(0, RPT // ECH, _zs, 0)
    plsc.subcore_barrier()

    def _run(g_hbm, gat_hbm, sca_hbm):
        def _start(j, rows, sem):
            pltpu.async_copy(g_hbm.at[esrc.at[j]], rows, sem)

        def _wait(rows, sem):
            pltpu.make_async_copy(g_hbm.at[esrc.at[0]], rows, sem).wait()

        def _scat(j, rows):
            pltpu.sync_copy(rows, acc.at[edst.at[j]], add=True)

        def _slice(t, carry0):
            wid = t * NS + s

            def _group(g, carry):
                pltpu.sync_copy(gat_hbm.at[wid, g], esrc)
                pltpu.sync_copy(sca_hbm.at[wid, g], edst)
                _start(0, rows0, sem0)

                def _body(i, carry2):
                    j0 = 2 * i
                    _start(j0 + 1, rows1, sem1)
                    _wait(rows0, sem0)
                    _scat(j0, rows0)

                    @pl.when(j0 + 2 < GCH)
                    def _():
                        _start(j0 + 2, rows0, sem0)

                    _wait(rows1, sem1)
                    _scat(j0 + 1, rows1)
                    return carry2

                lax.fori_loop(0, GCH // 2, _body, 0)
                return carry

            lax.fori_loop(0, NG, _group, 0)
            return carry0

        lax.fori_loop(0, NC, _slice, 0)

    @pl.when(c == 0)
    def _():
        _run(gin_hbm, src0_hbm, dstN_hbm)

    @pl.when(c == 1)
    def _():
        _run(gout_hbm, dst0_hbm, srcN_hbm)

    plsc.subcore_barrier()

    @pl.when(c == 0)
    def _():
        pltpu.sync_copy(acc.at[pl.ds(s * RPT, RPT)],
                        aggin_out.at[pl.ds(s * RPT, RPT)])

    @pl.when(c == 1)
    def _():
        pltpu.sync_copy(acc.at[pl.ds(s * RPT, RPT)],
                        aggout_out.at[pl.ds(s * RPT, RPT)])


BN = 1000
GRID = N // BN


def _mm(a, b):
    return jnp.dot(a, b, preferred_element_type=jnp.float32)


def _branch(cat, wl_ref, bl_ref, wc_ref, dis):
    acc = _mm(cat[0], wl_ref[0:C, :])
    for k in range(1, len(cat)):
        acc += _mm(cat[k], wl_ref[k * C:(k + 1) * C, :])
    p = jnp.maximum(acc + bl_ref[...], 0.0)
    return _mm(p, wc_ref[...]) * dis


def _fused0_body(h_ref, wli_ref, bli_ref, wci_ref, dis_ref,
                 wlo_ref, blo_ref, wco_ref, dist_ref, gi_ref, go_ref):
    cat = (h_ref[...],)
    gi_ref[...] = _branch(cat, wli_ref, bli_ref, wci_ref, dis_ref[...])
    go_ref[...] = _branch(cat, wlo_ref, blo_ref, wco_ref, dist_ref[...])


_wspec = [
    pl.BlockSpec((C, C), lambda i: (0, 0)),
    pl.BlockSpec((1, C), lambda i: (0, 0)),
    pl.BlockSpec((C, C), lambda i: (0, 0)),
    pl.BlockSpec((BN, 1), lambda i: (i, 0)),
]
_nspec = pl.BlockSpec((BN, C), lambda i: (i, 0))

_fused0 = pl.pallas_call(
    _fused0_body,
    grid=(GRID,),
    in_specs=[_nspec] + _wspec + _wspec,
    out_specs=(_nspec, _nspec),
    out_shape=(jax.ShapeDtypeStruct((N, C), jnp.float32),
               jax.ShapeDtypeStruct((N, C), jnp.float32)),
)


def _fused1_body(h_ref, ai_ref, gi0_ref, bci_ref, dis_ref,
                 ao_ref, go0_ref, bco_ref, dist_ref,
                 wli_ref, bli_ref, wci_ref, wlo_ref, blo_ref, wco_ref,
                 gi_ref, go_ref, xi_ref, xo_ref):
    dis = dis_ref[...]
    dist = dist_ref[...]
    x_in0 = jnp.maximum(
        (ai_ref[...] + gi0_ref[...]) * dis + bci_ref[...], 0.0)
    x_out0 = jnp.maximum(
        (ao_ref[...] + go0_ref[...]) * dist + bco_ref[...], 0.0)
    xi_ref[...] = x_in0
    xo_ref[...] = x_out0
    cat = (h_ref[...], x_in0, x_out0)
    gi_ref[...] = _branch(cat, wli_ref, bli_ref, wci_ref, dis)
    go_ref[...] = _branch(cat, wlo_ref, blo_ref, wco_ref, dist)


_w3spec = [
    pl.BlockSpec((3 * C, C), lambda i: (0, 0)),
    pl.BlockSpec((1, C), lambda i: (0, 0)),
    pl.BlockSpec((C, C), lambda i: (0, 0)),
]

_fused1 = pl.pallas_call(
    _fused1_body,
    grid=(GRID,),
    in_specs=([_nspec,
               _nspec, _nspec, pl.BlockSpec((1, C), lambda i: (0, 0)),
               pl.BlockSpec((BN, 1), lambda i: (i, 0)),
               _nspec, _nspec, pl.BlockSpec((1, C), lambda i: (0, 0)),
               pl.BlockSpec((BN, 1), lambda i: (i, 0))]
              + _w3spec + _w3spec),
    out_specs=(_nspec, _nspec, _nspec, _nspec),
    out_shape=tuple(jax.ShapeDtypeStruct((N, C), jnp.float32)
                    for _ in range(4)),
)


def _dis_body(degp_ref, degtp_ref, dis_ref, dist_ref):
    d = degp_ref[0:1, :] + degp_ref[1:2, :] + 1.0
    dis_ref[...] = lax.rsqrt(d)
    dt = degtp_ref[0:1, :] + degtp_ref[1:2, :] + 1.0
    dist_ref[...] = lax.rsqrt(dt)


_dis_call = pl.pallas_call(
    _dis_body,
    out_shape=(jax.ShapeDtypeStruct((1, NP), jnp.float32),
               jax.ShapeDtypeStruct((1, NP), jnp.float32)),
)


def _colsum_body(x0, x1, x2, ai_ref, gi_ref, bci_ref, dis_ref,
                 ao_ref, go_ref, bco_ref, dist_ref, out_ref):
    i = pl.program_id(0)

    @pl.when(i == 0)
    def _():
        out_ref[...] = jnp.zeros_like(out_ref)

    x_in1 = jnp.maximum(
        (ai_ref[...] + gi_ref[...]) * dis_ref[...] + bci_ref[...], 0.0)
    x_out1 = jnp.maximum(
        (ao_ref[...] + go_ref[...]) * dist_ref[...] + bco_ref[...], 0.0)
    for k, x in enumerate((x0[...], x1[...], x2[...], x_in1, x_out1)):
        out_ref[:, k * C:(k + 1) * C] += jnp.sum(x, axis=0, keepdims=True)


_colsum = pl.pallas_call(
    _colsum_body,
    grid=(GRID,),
    in_specs=[_nspec, _nspec, _nspec,
              _nspec, _nspec, pl.BlockSpec((1, C), lambda i: (0, 0)),
              pl.BlockSpec((BN, 1), lambda i: (i, 0)),
              _nspec, _nspec, pl.BlockSpec((1, C), lambda i: (0, 0)),
              pl.BlockSpec((BN, 1), lambda i: (i, 0))],
    out_specs=pl.BlockSpec((1, 5 * C), lambda i: (0, 0)),
    out_shape=jax.ShapeDtypeStruct((1, 5 * C), jnp.float32),
)


def _mlp_body(feat_ref, hw_ref, hb_ref, mw_ref, mb_ref, vw_ref, vb_ref,
              mean_ref, var_ref):
    f = feat_ref[...] * (1.0 / N)
    hdn = jnp.maximum(
        jnp.dot(f, hw_ref[...], preferred_element_type=jnp.float32)
        + hb_ref[...], 0.0)
    m = jnp.dot(hdn, mw_ref[...], preferred_element_type=jnp.float32) + mb_ref[...]
    mean_ref[...] = 2.0 * jnp.tanh(m)
    v = jnp.dot(hdn, vw_ref[...], preferred_element_type=jnp.float32) + vb_ref[...]
    var_ref[...] = 2.0 * jax.nn.sigmoid(v)


_mlp = pl.pallas_call(
    _mlp_body,
    out_shape=(jax.ShapeDtypeStruct((1, OUT), jnp.float32),
               jax.ShapeDtypeStruct((1, OUT), jnp.float32)),
)


def kernel(x, edge_index, edge_index_t, embed_W,
           li_W0, li_b0, lo_W0, lo_b0, ci_W0, ci_b0, co_W0, co_b0,
           li_W1, li_b1, lo_W1, lo_b1, ci_W1, ci_b1, co_W1, co_b1,
           hidden_W, hidden_b, mean_W, mean_b, var_W, var_b):
    npad = EP - E

    pad_scatter = N + jnp.arange(npad, dtype=jnp.int32) % (NP - N)
    pad_gather = jnp.arange(npad, dtype=jnp.int32) % N

    def pad_edges(ids, fill):
        return jnp.concatenate([ids, fill]).reshape(NW, NG, GCH, ECH)

    src0 = pad_edges(edge_index[0], pad_gather)
    dst0 = pad_edges(edge_index[1], pad_gather)
    srcN = pad_edges(edge_index[0], pad_scatter)
    dstN = pad_edges(edge_index[1], pad_scatter)
    xp = jnp.concatenate(
        [x, jnp.zeros((NP - N,), jnp.int32)]).reshape(NW, XCH, XE)
    zeros1 = jnp.zeros((NP,), jnp.float32)
    ones1 = jnp.ones((ECH,), jnp.float32)

    hp, degp, degtp = _sc_embed_deg(xp, srcN, dstN, embed_W, zeros1, ones1)
    h = hp[:N]

    dis2, dist2 = _dis_call(degp.reshape(NC, NP), degtp.reshape(NC, NP))
    dis = dis2.reshape(NP, 1)[:N]
    dist = dist2.reshape(NP, 1)[:N]

    def b2(b):
        return b.reshape(1, -1)

    g_in0, g_out0 = _fused0(h, li_W0, b2(li_b0), ci_W0, dis,
                            lo_W0, b2(lo_b0), co_W0, dist)
    agg_in0, agg_out0 = _sc_agg2(g_in0, g_out0, src0, dst0, srcN, dstN)

    g_in1, g_out1, x_in0, x_out0 = _fused1(
        h, agg_in0, g_in0, b2(ci_b0), dis,
        agg_out0, g_out0, b2(co_b0), dist,
        li_W1, b2(li_b1), ci_W1, lo_W1, b2(lo_b1), co_W1)
    agg_in1, agg_out1 = _sc_agg2(g_in1, g_out1, src0, dst0, srcN, dstN)

    feat = _colsum(h, x_in0, x_out0,
                   agg_in1, g_in1, b2(ci_b1), dis,
                   agg_out1, g_out1, b2(co_b1), dist)
    mean, var = _mlp(feat, hidden_W, b2(hidden_b), mean_W, b2(mean_b),
                     var_W, b2(var_b))
    return (mean.reshape(OUT), var.reshape(OUT))


# speedup vs baseline: 1.3069x; 1.0177x over previous
import functools
import jax
import jax.numpy as jnp
from jax import lax
from jax.experimental import pallas as pl
from jax.experimental.pallas import tpu as pltpu
from jax.experimental.pallas import tpu_sc as plsc

N = 10000
NP = 10240
E = 320000
C = 128
VOCAB = 100000
OUT = 16

NC = 2
NS = 16
NW = NC * NS
ECH = 128
EP = NW * 80 * ECH
EPT = EP // NW
NCH = EPT // ECH
RPT = NP // NS
DPT = NP // NS
XE = 80
XCH = 4
GCH = 40
NG = NCH // GCH

_sc_mesh = plsc.VectorSubcoreMesh(core_axis_name="c", subcore_axis_name="s")


@functools.partial(
    pl.kernel,
    out_type=(
        jax.ShapeDtypeStruct((NP, C), jnp.float32),
        jax.ShapeDtypeStruct((NC * NP,), jnp.float32),
        jax.ShapeDtypeStruct((NC * NP,), jnp.float32),
    ),
    mesh=_sc_mesh,
    scratch_types=[
        pltpu.VMEM((XCH, XE), jnp.int32),
        pltpu.VMEM((NG, GCH, ECH), jnp.int32),
        pltpu.VMEM((NG, GCH, ECH), jnp.int32),
        pltpu.VMEM((XE, C), jnp.float32),
        pltpu.VMEM((XE, C), jnp.float32),
        pltpu.VMEM((ECH,), jnp.float32),
        pltpu.VMEM_SHARED((NP,), jnp.float32),
        pltpu.VMEM_SHARED((NP,), jnp.float32),
        pltpu.SemaphoreType.DMA,
        pltpu.SemaphoreType.DMA,
        pltpu.SemaphoreType.DMA,
        pltpu.SemaphoreType.DMA,
    ],
)
def _sc_embed_deg(xp_hbm, src_hbm, dst_hbm, table_hbm, zeros1_hbm, ones_hbm,
                  h_out, degp_out, degtp_out,
                  xidx, esrc, edst, rows, rows2, ones, acc_d, acc_t,
                  sem, sem2, sem_d, sem_t):
    c = lax.axis_index("c")
    s = lax.axis_index("s")
    wid = c * NS + s

    pltpu.sync_copy(src_hbm.at[wid], esrc)
    pltpu.sync_copy(dst_hbm.at[wid], edst)
    pltpu.sync_copy(xp_hbm.at[wid], xidx)
    pltpu.sync_copy(ones_hbm, ones)
    pltpu.sync_copy(zeros1_hbm.at[pl.ds(s * DPT, DPT)], acc_d.at[pl.ds(s * DPT, DPT)])
    pltpu.sync_copy(zeros1_hbm.at[pl.ds(s * DPT, DPT)], acc_t.at[pl.ds(s * DPT, DPT)])
    plsc.subcore_barrier()

    def _hist_g(g, carry):
        def _hist(j, carry2):
            pltpu.async_copy(ones, acc_d.at[edst.at[g, j]], sem_d, add=True)
            pltpu.async_copy(ones, acc_t.at[esrc.at[g, j]], sem_t, add=True)

            @pl.when(jnp.logical_or(g > 0, j > 0))
            def _():
                pltpu.make_async_copy(ones, acc_d.at[edst.at[0, 0]], sem_d).wait()
                pltpu.make_async_copy(ones, acc_t.at[esrc.at[0, 0]], sem_t).wait()

            return carry2

        lax.fori_loop(0, GCH, _hist, 0)
        return carry

    lax.fori_loop(0, NG, _hist_g, 0)
    pltpu.make_async_copy(ones, acc_d.at[edst.at[0, 0]], sem_d).wait()
    pltpu.make_async_copy(ones, acc_t.at[esrc.at[0, 0]], sem_t).wait()

    bufs = (rows, rows2)
    sems = (sem, sem2)
    pltpu.async_copy(table_hbm.at[xidx.at[0]], rows, sem)
    for j in range(XCH):
        if j + 1 < XCH:
            pltpu.async_copy(table_hbm.at[xidx.at[j + 1]],
                             bufs[(j + 1) % 2], sems[(j + 1) % 2])
        pltpu.make_async_copy(table_hbm.at[xidx.at[0]],
                              bufs[j % 2], sems[j % 2]).wait()
        base = wid * (XCH * XE) + j * XE
        pltpu.sync_copy(bufs[j % 2], h_out.at[pl.ds(base, XE)])

    plsc.subcore_barrier()
    base = c * NP + s * DPT
    pltpu.sync_copy(acc_d.at[pl.ds(s * DPT, DPT)], degp_out.at[pl.ds(base, DPT)])
    pltpu.sync_copy(acc_t.at[pl.ds(s * DPT, DPT)], degtp_out.at[pl.ds(base, DPT)])


@functools.partial(
    pl.kernel,
    out_type=(jax.ShapeDtypeStruct((NP, C), jnp.float32),
              jax.ShapeDtypeStruct((NP, C), jnp.float32)),
    mesh=_sc_mesh,
    scratch_types=[
        pltpu.VMEM((GCH, ECH), jnp.int32),
        pltpu.VMEM((GCH, ECH), jnp.int32),
        pltpu.VMEM((ECH, C), jnp.float32),
        pltpu.VMEM((ECH, C), jnp.float32),
        pltpu.VMEM_SHARED((NP, C), jnp.float32),
        pltpu.SemaphoreType.DMA,
        pltpu.SemaphoreType.DMA,
    ],
)
def _sc_agg2(gin_hbm, gout_hbm, src0_hbm, dst0_hbm, srcN_hbm, dstN_hbm,
             aggin_out, aggout_out,
             esrc, edst, rows0, rows1, acc, sem0, sem1):
    c = lax.axis_index("c")
    s = lax.axis_index("s")

    def _z(i, carry):
        def _zk(k, carry2):
            rows0[i, pl.ds(k * 16, 16)] = jnp.zeros((16,), jnp.float32)
            return carry2

        lax.fori_loop(0, C // 16, _zk, 0)
        return carry

    lax.fori_loop(0, ECH, _z, 0)

    def _zs(i, carry):
        pltpu.sync_copy(rows0, acc.at[pl.ds(s * RPT + i * ECH, ECH)])
        return carry

    lax.fori_loop(0, RPT // ECH, _zs, 0)
    plsc.subcore_barrier()

    def _run(g_hbm, gat_hbm, sca_hbm):
        def _start(j, rows, sem):
            pltpu.async_copy(g_hbm.at[esrc.at[j]], rows, sem)

        def _wait(rows, sem):
            pltpu.make_async_copy(g_hbm.at[esrc.at[0]], rows, sem).wait()

        def _scat(j, rows):
            pltpu.sync_copy(rows, acc.at[edst.at[j]], add=True)

        def _slice(t, carry0):
            wid = t * NS + s

            def _group(g, carry):
                pltpu.sync_copy(gat_hbm.at[wid, g], esrc)
                pltpu.sync_copy(sca_hbm.at[wid, g], edst)
                _start(0, rows0, sem0)

                def _body(i, carry2):
                    j0 = 2 * i
                    _start(j0 + 1, rows1, sem1)
                    _wait(rows0, sem0)
                    _scat(j0, rows0)

                    @pl.when(j0 + 2 < GCH)
                    def _():
                        _start(j0 + 2, rows0, sem0)

                    _wait(rows1, sem1)
                    _scat(j0 + 1, rows1)
                    return carry2

                lax.fori_loop(0, GCH // 2, _body, 0)
                return carry

            lax.fori_loop(0, NG, _group, 0)
            return carry0

        lax.fori_loop(0, NC, _slice, 0)

    @pl.when(c == 0)
    def _():
        _run(gin_hbm, src0_hbm, dstN_hbm)

    @pl.when(c == 1)
    def _():
        _run(gout_hbm, dst0_hbm, srcN_hbm)

    plsc.subcore_barrier()

    @pl.when(c == 0)
    def _():
        pltpu.sync_copy(acc.at[pl.ds(s * RPT, RPT)],
                        aggin_out.at[pl.ds(s * RPT, RPT)])

    @pl.when(c == 1)
    def _():
        pltpu.sync_copy(acc.at[pl.ds(s * RPT, RPT)],
                        aggout_out.at[pl.ds(s * RPT, RPT)])


BN = 1000
GRID = N // BN


def _mm(a, b):
    return jnp.dot(a, b, preferred_element_type=jnp.float32)


def _branch(cat, wl_ref, bl_ref, wc_ref, dis):
    acc = _mm(cat[0], wl_ref[0:C, :])
    for k in range(1, len(cat)):
        acc += _mm(cat[k], wl_ref[k * C:(k + 1) * C, :])
    p = jnp.maximum(acc + bl_ref[...], 0.0)
    return _mm(p, wc_ref[...]) * dis


def _fused0_body(h_ref, wli_ref, bli_ref, wci_ref, dis_ref,
                 wlo_ref, blo_ref, wco_ref, dist_ref, gi_ref, go_ref):
    cat = (h_ref[...],)
    gi_ref[...] = _branch(cat, wli_ref, bli_ref, wci_ref, dis_ref[...])
    go_ref[...] = _branch(cat, wlo_ref, blo_ref, wco_ref, dist_ref[...])


_wspec = [
    pl.BlockSpec((C, C), lambda i: (0, 0)),
    pl.BlockSpec((1, C), lambda i: (0, 0)),
    pl.BlockSpec((C, C), lambda i: (0, 0)),
    pl.BlockSpec((BN, 1), lambda i: (i, 0)),
]
_nspec = pl.BlockSpec((BN, C), lambda i: (i, 0))

_fused0 = pl.pallas_call(
    _fused0_body,
    grid=(GRID,),
    in_specs=[_nspec] + _wspec + _wspec,
    out_specs=(_nspec, _nspec),
    out_shape=(jax.ShapeDtypeStruct((N, C), jnp.float32),
               jax.ShapeDtypeStruct((N, C), jnp.float32)),
)


def _fused1_body(h_ref, ai_ref, gi0_ref, bci_ref, dis_ref,
                 ao_ref, go0_ref, bco_ref, dist_ref,
                 wli_ref, bli_ref, wci_ref, wlo_ref, blo_ref, wco_ref,
                 gi_ref, go_ref, xi_ref, xo_ref):
    dis = dis_ref[...]
    dist = dist_ref[...]
    x_in0 = jnp.maximum(
        (ai_ref[...] + gi0_ref[...]) * dis + bci_ref[...], 0.0)
    x_out0 = jnp.maximum(
        (ao_ref[...] + go0_ref[...]) * dist + bco_ref[...], 0.0)
    xi_ref[...] = x_in0
    xo_ref[...] = x_out0
    cat = (h_ref[...], x_in0, x_out0)
    gi_ref[...] = _branch(cat, wli_ref, bli_ref, wci_ref, dis)
    go_ref[...] = _branch(cat, wlo_ref, blo_ref, wco_ref, dist)


_w3spec = [
    pl.BlockSpec((3 * C, C), lambda i: (0, 0)),
    pl.BlockSpec((1, C), lambda i: (0, 0)),
    pl.BlockSpec((C, C), lambda i: (0, 0)),
]

_fused1 = pl.pallas_call(
    _fused1_body,
    grid=(GRID,),
    in_specs=([_nspec,
               _nspec, _nspec, pl.BlockSpec((1, C), lambda i: (0, 0)),
               pl.BlockSpec((BN, 1), lambda i: (i, 0)),
               _nspec, _nspec, pl.BlockSpec((1, C), lambda i: (0, 0)),
               pl.BlockSpec((BN, 1), lambda i: (i, 0))]
              + _w3spec + _w3spec),
    out_specs=(_nspec, _nspec, _nspec, _nspec),
    out_shape=tuple(jax.ShapeDtypeStruct((N, C), jnp.float32)
                    for _ in range(4)),
)


def _dis_body(degp_ref, degtp_ref, dis_ref, dist_ref):
    d = degp_ref[0:1, :] + degp_ref[1:2, :] + 1.0
    dis_ref[...] = lax.rsqrt(d)
    dt = degtp_ref[0:1, :] + degtp_ref[1:2, :] + 1.0
    dist_ref[...] = lax.rsqrt(dt)


_dis_call = pl.pallas_call(
    _dis_body,
    out_shape=(jax.ShapeDtypeStruct((1, NP), jnp.float32),
               jax.ShapeDtypeStruct((1, NP), jnp.float32)),
)


def _colsum_body(x0, x1, x2, ai_ref, gi_ref, bci_ref, dis_ref,
                 ao_ref, go_ref, bco_ref, dist_ref, out_ref):
    i = pl.program_id(0)

    @pl.when(i == 0)
    def _():
        out_ref[...] = jnp.zeros_like(out_ref)

    x_in1 = jnp.maximum(
        (ai_ref[...] + gi_ref[...]) * dis_ref[...] + bci_ref[...], 0.0)
    x_out1 = jnp.maximum(
        (ao_ref[...] + go_ref[...]) * dist_ref[...] + bco_ref[...], 0.0)
    for k, x in enumerate((x0[...], x1[...], x2[...], x_in1, x_out1)):
        out_ref[:, k * C:(k + 1) * C] += jnp.sum(x, axis=0, keepdims=True)


_colsum = pl.pallas_call(
    _colsum_body,
    grid=(GRID,),
    in_specs=[_nspec, _nspec, _nspec,
              _nspec, _nspec, pl.BlockSpec((1, C), lambda i: (0, 0)),
              pl.BlockSpec((BN, 1), lambda i: (i, 0)),
              _nspec, _nspec, pl.BlockSpec((1, C), lambda i: (0, 0)),
              pl.BlockSpec((BN, 1), lambda i: (i, 0))],
    out_specs=pl.BlockSpec((1, 5 * C), lambda i: (0, 0)),
    out_shape=jax.ShapeDtypeStruct((1, 5 * C), jnp.float32),
)


def _mlp_body(feat_ref, hw_ref, hb_ref, mw_ref, mb_ref, vw_ref, vb_ref,
              mean_ref, var_ref):
    f = feat_ref[...] * (1.0 / N)
    hdn = jnp.maximum(
        jnp.dot(f, hw_ref[...], preferred_element_type=jnp.float32)
        + hb_ref[...], 0.0)
    m = jnp.dot(hdn, mw_ref[...], preferred_element_type=jnp.float32) + mb_ref[...]
    mean_ref[...] = 2.0 * jnp.tanh(m)
    v = jnp.dot(hdn, vw_ref[...], preferred_element_type=jnp.float32) + vb_ref[...]
    var_ref[...] = 2.0 * jax.nn.sigmoid(v)


_mlp = pl.pallas_call(
    _mlp_body,
    out_shape=(jax.ShapeDtypeStruct((1, OUT), jnp.float32),
               jax.ShapeDtypeStruct((1, OUT), jnp.float32)),
)


def kernel(x, edge_index, edge_index_t, embed_W,
           li_W0, li_b0, lo_W0, lo_b0, ci_W0, ci_b0, co_W0, co_b0,
           li_W1, li_b1, lo_W1, lo_b1, ci_W1, ci_b1, co_W1, co_b1,
           hidden_W, hidden_b, mean_W, mean_b, var_W, var_b):
    npad = EP - E

    pad_scatter = N + jnp.arange(npad, dtype=jnp.int32) % (NP - N)
    pad_gather = jnp.arange(npad, dtype=jnp.int32) % N

    def pad_edges(ids, fill):
        return jnp.concatenate([ids, fill]).reshape(NW, NG, GCH, ECH)

    src0 = pad_edges(edge_index[0], pad_gather)
    dst0 = pad_edges(edge_index[1], pad_gather)
    srcN = pad_edges(edge_index[0], pad_scatter)
    dstN = pad_edges(edge_index[1], pad_scatter)
    xp = jnp.concatenate(
        [x, jnp.zeros((NP - N,), jnp.int32)]).reshape(NW, XCH, XE)
    zeros1 = jnp.zeros((NP,), jnp.float32)
    ones1 = jnp.ones((ECH,), jnp.float32)

    hp, degp, degtp = _sc_embed_deg(xp, srcN, dstN, embed_W, zeros1, ones1)
    h = hp[:N]

    dis2, dist2 = _dis_call(degp.reshape(NC, NP), degtp.reshape(NC, NP))
    dis = dis2.reshape(NP, 1)[:N]
    dist = dist2.reshape(NP, 1)[:N]

    def b2(b):
        return b.reshape(1, -1)

    g_in0, g_out0 = _fused0(h, li_W0, b2(li_b0), ci_W0, dis,
                            lo_W0, b2(lo_b0), co_W0, dist)
    agg_in0, agg_out0 = _sc_agg2(g_in0, g_out0, src0, dst0, srcN, dstN)

    g_in1, g_out1, x_in0, x_out0 = _fused1(
        h, agg_in0, g_in0, b2(ci_b0), dis,
        agg_out0, g_out0, b2(co_b0), dist,
        li_W1, b2(li_b1), ci_W1, lo_W1, b2(lo_b1), co_W1)
    agg_in1, agg_out1 = _sc_agg2(g_in1, g_out1, src0, dst0, srcN, dstN)

    feat = _colsum(h, x_in0, x_out0,
                   agg_in1, g_in1, b2(ci_b1), dis,
                   agg_out1, g_out1, b2(co_b1), dist)
    mean, var = _mlp(feat, hidden_W, b2(hidden_b), mean_W, b2(mean_b),
                     var_W, b2(var_b))
    return (mean.reshape(OUT), var.reshape(OUT))
